# baseline (speedup 1.0000x reference)
import functools

import jax
import jax.numpy as jnp
from jax import lax
from jax.experimental import pallas as pl
from jax.experimental.pallas import tpu as pltpu
from jax.experimental.pallas import tpu_sc as plsc

N = 100000
D = 128
B = 16384
NW = 32
OWN = 3136
WIN = 128
NCH = B // 16


def _body(mem_h, lu_h, idx_h, val_h, vlu_h,
          nm_h, nlu_h, gm_h, glu_h,
          idx_v, pos_v, bl_v, nl_v, wl_v, row_v, lub_v, sem):
    wid = lax.axis_index("s") * 2 + lax.axis_index("c")
    base = wid * OWN
    own = jnp.minimum(OWN, N - base)

    pltpu.sync_copy(idx_h, idx_v)

    neg1 = jnp.full((16,), -1, jnp.int32)

    def init_body(c, carry):
        pos_v[pl.ds(c * 16, 16)] = neg1
        return carry

    lax.fori_loop(0, OWN // 16, init_body, 0)

    iota = lax.iota(jnp.int32, 16)

    def chunk(c, k):
        v = idx_v[pl.ds(c * 16, 16)]
        rel = v - base
        inr = (rel >= 0) & (rel < own)
        anyin = jnp.max(inr.astype(jnp.int32))

        def active(k):
            relc = jnp.clip(rel, 0, OWN - 1)
            j = c * 16 + iota

            def wcond(nb):
                return nb > 0

            def wbody(nb):
                w = plsc.load_gather(pos_v, [relc], mask=inr)
                better = inr & (j > w)
                plsc.store_scatter(pos_v, [relc], j, mask=better)
                return jnp.sum(better.astype(jnp.int32))

            lax.while_loop(wcond, wbody, jnp.int32(1))

            cnt = inr.astype(jnp.int32)
            incl = plsc.cumsum(cnt)
            tgt = k + incl - 1
            tr = tgt >> 7
            tc = tgt & 127
            plsc.store_scatter(bl_v, [tr, tc], j, mask=inr)
            plsc.store_scatter(nl_v, [tr, tc], v, mask=inr)
            return k + jnp.sum(cnt)

        return lax.cond(anyin > 0, active, lambda k: k, k)

    K = lax.fori_loop(0, NCH, chunk, jnp.int32(0))

    nq = (K + 15) >> 4

    def fillw(q, carry):
        flat = q * 16 + iota
        m = flat < K
        fr = flat >> 7
        fc = flat & 127
        nodes = plsc.load_gather(nl_v, [fr, fc], mask=m)
        rel = jnp.clip(nodes - base, 0, OWN - 1)
        w = plsc.load_gather(pos_v, [rel], mask=m)
        plsc.store_scatter(wl_v, [fr, fc], w, mask=m)
        return carry

    lax.fori_loop(0, nq, fillw, 0)

    nwin = (K + 127) >> 7
    lim = nwin * 128
    zero16 = jnp.zeros((16,), jnp.int32)
    e_b = plsc.load_gather(bl_v, [zero16, zero16])
    e_n = plsc.load_gather(nl_v, [zero16, zero16])
    e_w = plsc.load_gather(wl_v, [zero16, zero16])

    def padp(p, carry):
        flat = K + p * 16 + iota
        m = flat < lim
        fr = flat >> 7
        fc = flat & 127
        plsc.store_scatter(bl_v, [fr, fc], e_b, mask=m)
        plsc.store_scatter(nl_v, [fr, fc], e_n, mask=m)
        plsc.store_scatter(wl_v, [fr, fc], e_w, mask=m)
        return carry

    lax.fori_loop(0, 8, padp, 0)

    nwc = (own + WIN - 1) >> 7

    def cpy(w, carry):
        start = base + jnp.minimum(w * WIN, own - WIN)
        pltpu.sync_copy(mem_h.at[pl.ds(start, WIN)], row_v)
        pltpu.sync_copy(row_v, nm_h.at[pl.ds(start, WIN)])
        pltpu.sync_copy(lu_h.at[pl.ds(start, WIN)], lub_v)
        pltpu.sync_copy(lub_v, nlu_h.at[pl.ds(start, WIN)])
        return carry

    lax.fori_loop(0, nwc, cpy, 0)

    def patch(w, carry):
        pltpu.async_copy(val_h.at[wl_v.at[w]], row_v, sem).wait()
        pltpu.sync_copy(row_v, nm_h.at[nl_v.at[w]])
        pltpu.sync_copy(row_v, gm_h.at[bl_v.at[w]])
        pltpu.async_copy(vlu_h.at[wl_v.at[w]], lub_v, sem).wait()
        pltpu.sync_copy(lub_v, nlu_h.at[nl_v.at[w]])
        pltpu.sync_copy(lub_v, glu_h.at[bl_v.at[w]])
        return carry

    lax.fori_loop(0, nwin, patch, 0)


@jax.jit
def kernel(memory, last_update, idx, value_memory, value_last_update):
    idx = idx.astype(jnp.int32)
    run = pl.kernel(
        _body,
        out_type=(
            jax.ShapeDtypeStruct((N, D), jnp.float32),
            jax.ShapeDtypeStruct((N,), jnp.float32),
            jax.ShapeDtypeStruct((B, D), jnp.float32),
            jax.ShapeDtypeStruct((B,), jnp.float32),
        ),
        mesh=plsc.VectorSubcoreMesh(core_axis_name="c", subcore_axis_name="s"),
        compiler_params=pltpu.CompilerParams(needs_layout_passes=False),
        scratch_types=[
            pltpu.VMEM((B,), jnp.int32),
            pltpu.VMEM((OWN,), jnp.int32),
            pltpu.VMEM((128, 128), jnp.int32),
            pltpu.VMEM((128, 128), jnp.int32),
            pltpu.VMEM((128, 128), jnp.int32),
            pltpu.VMEM((WIN, D), jnp.float32),
            pltpu.VMEM((WIN,), jnp.float32),
            pltpu.SemaphoreType.DMA,
        ],
    )
    return run(memory, last_update, idx, value_memory, value_last_update)
